# baseline (speedup 1.0000x reference)
import functools

import jax
import jax.numpy as jnp
from jax import lax
from jax.experimental import pallas as pl
from jax.experimental.pallas import tpu as pltpu
from jax.experimental.pallas import tpu_sc as plsc

N = 10000
E = 160000
D = 256
R = 16
NBASE = 4
HALF = 128
NSC = 2
NTILES = 16
EPT = E // (NSC * NTILES)
CH = 40
NCH = EPT // CH
ZROWS_PT = 624
ZTAIL = N - NTILES * ZROWS_PT


def _xw_half_body(wc_ref, x_ref, basis_ref, out_ref):
    w = (wc_ref[0, 0, 0] * basis_ref[0]
         + wc_ref[0, 0, 1] * basis_ref[1]
         + wc_ref[0, 0, 2] * basis_ref[2]
         + wc_ref[0, 0, 3] * basis_ref[3])
    out_ref[...] = jnp.dot(x_ref[...], w.astype(jnp.bfloat16),
                           preferred_element_type=jnp.float32)


def _compute_xw_half(x_bf, basis_bf, w_comp3, h):
    return pl.pallas_call(
        _xw_half_body,
        grid=(R,),
        in_specs=[
            pl.BlockSpec((1, 1, NBASE), lambda r: (r, 0, 0)),
            pl.BlockSpec((N, D), lambda r: (0, 0)),
            pl.BlockSpec((NBASE, D, HALF), lambda r, h=h: (0, 0, h)),
        ],
        out_specs=pl.BlockSpec((N, HALF), lambda r: (r, 0)),
        out_shape=jax.ShapeDtypeStruct((R * N, HALF), jnp.float32),
    )(w_comp3, x_bf, basis_bf)


_sc_mesh = plsc.VectorSubcoreMesh(core_axis_name="c", subcore_axis_name="s")


@functools.partial(
    pl.kernel,
    out_type=jax.ShapeDtypeStruct((NSC, N, HALF), jnp.float32),
    mesh=_sc_mesh,
    scratch_types=[
        pltpu.VMEM((EPT,), jnp.int32),
        pltpu.VMEM((EPT,), jnp.int32),
        pltpu.VMEM((2, CH, HALF), jnp.float32),
        pltpu.VMEM_SHARED((N, HALF), jnp.float32),
        pltpu.SemaphoreType.DMA,
        pltpu.SemaphoreType.DMA,
    ],
)
def _sc_scatter(xw_hbm, eid_hbm, dst_hbm, out_hbm,
                dst_v, gid_v, rows_v, agg_sh, sem0, sem1):
    c = lax.axis_index("c")
    s = lax.axis_index("s")

    pltpu.sync_copy(eid_hbm.at[c, s], gid_v)
    pltpu.sync_copy(dst_hbm.at[c, s], dst_v)

    def zfill_body(k, _):
        def lane_body(j, _):
            rows_v[0, k, pl.ds(j * 16, 16)] = jnp.zeros((16,), jnp.float32)
            return 0
        lax.fori_loop(0, HALF // 16, lane_body, 0)
        return 0

    lax.fori_loop(0, CH, zfill_body, 0)
    base_row = s * ZROWS_PT
    for off in range(0, 600, 40):
        pltpu.sync_copy(rows_v.at[0],
                        agg_sh.at[pl.ds(base_row + off, CH)])
    pltpu.sync_copy(rows_v.at[0, pl.ds(0, 24)],
                    agg_sh.at[pl.ds(base_row + 600, 24)])

    @pl.when(s == NTILES - 1)
    def _():
        pltpu.sync_copy(rows_v.at[0, pl.ds(0, ZTAIL)],
                        agg_sh.at[pl.ds(NTILES * ZROWS_PT, ZTAIL)])

    plsc.subcore_barrier()

    sems = (sem0, sem1)

    def gather(k, buf):
        return pltpu.make_async_copy(
            xw_hbm.at[gid_v.at[pl.ds(k * CH, CH)]], rows_v.at[buf],
            sems[buf])

    def scatter(k, buf):
        pltpu.sync_copy(rows_v.at[buf],
                        agg_sh.at[dst_v.at[pl.ds(k * CH, CH)]], add=True)

    gather(0, 0).start()
    gather(1, 1).start()

    def main_body(g, _):
        k0 = 2 * g
        gather(k0, 0).wait()
        scatter(k0, 0)
        gather(k0 + 2, 0).start()

        k1 = k0 + 1
        gather(k1, 1).wait()
        scatter(k1, 1)

        @pl.when(k1 + 2 < NCH)
        def _():
            gather(k1 + 2, 1).start()
        return 0

    lax.fori_loop(0, (NCH - 1) // 2, main_body, 0)
    klast = NCH - 1
    gather(klast, 0).wait()
    scatter(klast, 0)

    plsc.subcore_barrier()

    pltpu.sync_copy(agg_sh.at[pl.ds(s * ZROWS_PT, ZROWS_PT)],
                    out_hbm.at[c, pl.ds(s * ZROWS_PT, ZROWS_PT)])

    @pl.when(s == NTILES - 1)
    def _():
        pltpu.sync_copy(agg_sh.at[pl.ds(NTILES * ZROWS_PT, ZTAIL)],
                        out_hbm.at[c, pl.ds(NTILES * ZROWS_PT, ZTAIL)])


def _mlp_body(a0_ref, a1_ref, x_ref, lw_ref, hb_ref,
              w1_ref, b1_ref, w2_ref, b2_ref, out_ref):
    agg = jnp.concatenate(
        [a0_ref[0] + a0_ref[1], a1_ref[0] + a1_ref[1]], axis=1)
    h = agg + jnp.dot(x_ref[...], lw_ref[...],
                      preferred_element_type=jnp.float32) + hb_ref[...]
    h = jnp.maximum(
        jnp.dot(h, w1_ref[...], preferred_element_type=jnp.float32)
        + b1_ref[...], 0.0)
    out_ref[...] = jnp.maximum(
        jnp.dot(h, w2_ref[...], preferred_element_type=jnp.float32)
        + b2_ref[...], 0.0)


def _mlp(p0, p1, x, loop_weight, h_bias, W1, b1, W2, b2):
    mat = lambda: pl.BlockSpec((D, D), lambda: (0, 0))
    vec = lambda: pl.BlockSpec((1, D), lambda: (0, 0))
    ph = lambda: pl.BlockSpec((NSC, N, HALF), lambda: (0, 0, 0))
    return pl.pallas_call(
        _mlp_body,
        in_specs=[
            ph(), ph(),
            pl.BlockSpec((N, D), lambda: (0, 0)),
            mat(), vec(), mat(), vec(), mat(), vec(),
        ],
        out_specs=pl.BlockSpec((N, D), lambda: (0, 0)),
        out_shape=jax.ShapeDtypeStruct((N, D), jnp.float32),
    )(p0, p1, x, loop_weight, h_bias.reshape(1, D), W1,
      b1.reshape(1, D), W2, b2.reshape(1, D))


def kernel(x, edge_index, etypes, basis, w_comp, loop_weight, h_bias,
           W1, b1, W2, b2):
    x_bf = x.astype(jnp.bfloat16)
    basis_bf = basis.astype(jnp.bfloat16)
    w_comp3 = w_comp.reshape(R, 1, NBASE)
    eid3 = (etypes * N + edge_index[0]).reshape(NSC, NTILES, EPT)
    dst3 = edge_index[1].reshape(NSC, NTILES, EPT)

    xw0 = _compute_xw_half(x_bf, basis_bf, w_comp3, 0)
    p0 = _sc_scatter(xw0, eid3, dst3)
    xw1 = _compute_xw_half(x_bf, basis_bf, w_comp3, 1)
    p1 = _sc_scatter(xw1, eid3, dst3)
    return _mlp(p0, p1, x, loop_weight, h_bias, W1, b1, W2, b2)


# speedup vs baseline: 1.2230x; 1.2230x over previous
import functools

import jax
import jax.numpy as jnp
from jax import lax
from jax.experimental import pallas as pl
from jax.experimental.pallas import tpu as pltpu
from jax.experimental.pallas import tpu_sc as plsc

N = 10000
E = 160000
D = 256
R = 16
NBASE = 4
HALF = 128
NSC = 2
NTILES = 16
EPT = E // NTILES
CH = 80
NCH = EPT // CH
ZROWS_PT = 624
ZTAIL = N - NTILES * ZROWS_PT

BN = 10000
NB = N // BN


def _xw_body(wc_ref, x_ref, basis_ref, out_ref, wall_ref):
    i = pl.program_id(0)
    r = pl.program_id(1)

    @pl.when(i == 0)
    def _():
        w = (wc_ref[0, 0, 0] * basis_ref[0]
             + wc_ref[0, 0, 1] * basis_ref[1]
             + wc_ref[0, 0, 2] * basis_ref[2]
             + wc_ref[0, 0, 3] * basis_ref[3])
        wall_ref[r] = w.astype(jnp.bfloat16)

    h = jnp.dot(x_ref[...], wall_ref[r],
                preferred_element_type=jnp.float32)
    for c in range(NSC):
        out_ref[c] = h[:, c * HALF:(c + 1) * HALF]


def _compute_xw(x, basis, w_comp):
    return pl.pallas_call(
        _xw_body,
        grid=(NB, R),
        in_specs=[
            pl.BlockSpec((1, 1, NBASE), lambda i, r: (r, 0, 0)),
            pl.BlockSpec((BN, D), lambda i, r: (i, 0)),
            pl.BlockSpec((NBASE, D, D), lambda i, r: (0, 0, 0)),
        ],
        out_specs=pl.BlockSpec(
            (NSC, BN, HALF), lambda i, r: (0, r * NB + i, 0)),
        out_shape=jax.ShapeDtypeStruct((NSC, R * N, HALF), jnp.float32),
        scratch_shapes=[pltpu.VMEM((R, D, D), jnp.bfloat16)],
    )(w_comp.reshape(R, 1, NBASE), x.astype(jnp.bfloat16),
      basis.astype(jnp.bfloat16))


_sc_mesh = plsc.VectorSubcoreMesh(core_axis_name="c", subcore_axis_name="s")


@functools.partial(
    pl.kernel,
    out_type=jax.ShapeDtypeStruct((NSC, N, HALF), jnp.float32),
    mesh=_sc_mesh,
    scratch_types=[
        pltpu.VMEM((EPT,), jnp.int32),
        pltpu.VMEM((EPT,), jnp.int32),
        pltpu.VMEM((2, CH, HALF), jnp.float32),
        pltpu.VMEM_SHARED((N, HALF), jnp.float32),
        pltpu.SemaphoreType.DMA,
        pltpu.SemaphoreType.DMA,
    ],
)
def _sc_scatter(xw_hbm, eid_hbm, dst_hbm, out_hbm,
                dst_v, gid_v, rows_v, agg_sh, sem0, sem1):
    c = lax.axis_index("c")
    s = lax.axis_index("s")

    pltpu.sync_copy(eid_hbm.at[s], gid_v)
    pltpu.sync_copy(dst_hbm.at[s], dst_v)

    def zfill_body(k, _):
        def lane_body(j, _):
            rows_v[0, k, pl.ds(j * 16, 16)] = jnp.zeros((16,), jnp.float32)
            return 0
        lax.fori_loop(0, HALF // 16, lane_body, 0)
        return 0

    lax.fori_loop(0, CH, zfill_body, 0)
    base_row = s * ZROWS_PT
    for off, nrows in ((0, 80), (80, 80), (160, 80), (240, 80),
                       (320, 80), (400, 80), (480, 80), (560, 64)):
        pltpu.sync_copy(rows_v.at[0, pl.ds(0, nrows)],
                        agg_sh.at[pl.ds(base_row + off, nrows)])

    @pl.when(s == NTILES - 1)
    def _():
        pltpu.sync_copy(rows_v.at[0, pl.ds(0, ZTAIL)],
                        agg_sh.at[pl.ds(NTILES * ZROWS_PT, ZTAIL)])

    base = c * (R * N)

    def idx_body(j, _):
        sl = pl.ds(j * 16, 16)
        gid_v[sl] = gid_v[sl] + base
        return 0

    lax.fori_loop(0, EPT // 16, idx_body, 0)
    plsc.subcore_barrier()

    sems = (sem0, sem1)

    def gather(k, buf):
        return pltpu.make_async_copy(
            xw_hbm.at[gid_v.at[pl.ds(k * CH, CH)]], rows_v.at[buf],
            sems[buf])

    def scatter(k, buf):
        pltpu.sync_copy(rows_v.at[buf],
                        agg_sh.at[dst_v.at[pl.ds(k * CH, CH)]], add=True)

    gather(0, 0).start()
    gather(1, 1).start()

    def main_body(g, _):
        k0 = 2 * g
        gather(k0, 0).wait()
        scatter(k0, 0)
        gather(k0 + 2, 0).start()

        k1 = k0 + 1
        gather(k1, 1).wait()
        scatter(k1, 1)

        @pl.when(k1 + 2 < NCH)
        def _():
            gather(k1 + 2, 1).start()
        return 0

    lax.fori_loop(0, (NCH - 1) // 2, main_body, 0)
    klast = NCH - 1
    gather(klast, 0).wait()
    scatter(klast, 0)

    plsc.subcore_barrier()

    pltpu.sync_copy(agg_sh.at[pl.ds(s * ZROWS_PT, ZROWS_PT)],
                    out_hbm.at[c, pl.ds(s * ZROWS_PT, ZROWS_PT)])

    @pl.when(s == NTILES - 1)
    def _():
        pltpu.sync_copy(agg_sh.at[pl.ds(NTILES * ZROWS_PT, ZTAIL)],
                        out_hbm.at[c, pl.ds(NTILES * ZROWS_PT, ZTAIL)])


def _mlp_body(a0_ref, a1_ref, x_ref, lw_ref, hb_ref,
              w1_ref, b1_ref, w2_ref, b2_ref, out_ref):
    agg = jnp.concatenate([a0_ref[0], a1_ref[0]], axis=1)
    h = agg + jnp.dot(x_ref[...], lw_ref[...],
                      preferred_element_type=jnp.float32) + hb_ref[...]
    h = jnp.maximum(
        jnp.dot(h, w1_ref[...], preferred_element_type=jnp.float32)
        + b1_ref[...], 0.0)
    out_ref[...] = jnp.maximum(
        jnp.dot(h, w2_ref[...], preferred_element_type=jnp.float32)
        + b2_ref[...], 0.0)


def _mlp(agg, x, loop_weight, h_bias, W1, b1, W2, b2):
    mat = lambda: pl.BlockSpec((D, D), lambda i: (0, 0))
    vec = lambda: pl.BlockSpec((1, D), lambda i: (0, 0))
    ah = lambda c: pl.BlockSpec((1, BN, HALF), lambda i, c=c: (c, i, 0))
    return pl.pallas_call(
        _mlp_body,
        grid=(NB,),
        in_specs=[
            ah(0), ah(1),
            pl.BlockSpec((BN, D), lambda i: (i, 0)),
            mat(), vec(), mat(), vec(), mat(), vec(),
        ],
        out_specs=pl.BlockSpec((BN, D), lambda i: (i, 0)),
        out_shape=jax.ShapeDtypeStruct((N, D), jnp.float32),
    )(agg, agg, x, loop_weight, h_bias.reshape(1, D), W1,
      b1.reshape(1, D), W2, b2.reshape(1, D))


def kernel(x, edge_index, etypes, basis, w_comp, loop_weight, h_bias,
           W1, b1, W2, b2):
    xw = _compute_xw(x, basis, w_comp).reshape(NSC * R * N, HALF)
    eid2 = (etypes * N + edge_index[0]).reshape(NTILES, EPT)
    dst2 = edge_index[1].reshape(NTILES, EPT)
    agg = _sc_scatter(xw, eid2, dst2)
    return _mlp(agg, x, loop_weight, h_bias, W1, b1, W2, b2)
